# baseline (device time: 67535 ns/iter reference)
import jax
import jax.numpy as jnp
from jax import lax
from jax.experimental import pallas as pl
from jax.experimental.pallas import tpu as pltpu

N_DEV = 8


def kernel(x, router_W, route_idx, expert_W, shared_W):
    n_tok, d = x.shape
    n_exp = router_W.shape[1]
    e_loc, _, h = expert_W.shape
    chunk = n_tok // N_DEV
    h2 = h // 2

    def body(x_ref, rw_ref, idx_ref, ew_ref, sw_ref, out_ref,
             partial_ref, rA0, rA1, rA2, rB0, rB1, rB2, psel_ref,
             A_s, A_r, B_s, B_r, aA_s, aA_r, aB_s, aB_r):
        my = lax.axis_index("i")
        ell = my ^ ((my >> 1) & 1)
        b1 = ell & 1
        b2 = (ell >> 1) & 1
        b4 = (ell >> 2) & 1

        def logi(l):
            return l ^ ((l >> 1) & 1)

        nx = logi(ell ^ 1)
        ny = logi(ell ^ 2)
        nz = logi(ell ^ 4)

        def rows(c, n=1):
            return pl.ds(c * chunk, n * chunk)

        A = pl.ds(0, h2)
        B = pl.ds(h2, h2)

        def mk(src, dst, ssem, rsem, dev):
            return pltpu.make_async_remote_copy(
                src_ref=src, dst_ref=dst, send_sem=ssem, recv_sem=rsem,
                device_id=(dev,), device_id_type=pl.DeviceIdType.MESH)

        xv = x_ref[:, :]

        scores = jnp.dot(xv, rw_ref[:, :], preferred_element_type=jnp.float32)
        m = jnp.max(scores, axis=-1, keepdims=True)
        p = jnp.exp(scores - m)
        probs = p / jnp.sum(p, axis=-1, keepdims=True)
        ridx = idx_ref[:, 0:1]
        e_ids = lax.broadcasted_iota(jnp.int32, (n_tok, n_exp), 1)
        p_sel = jnp.sum(jnp.where(e_ids == ridx, probs, 0.0),
                        axis=1, keepdims=True)

        psel_ref[:, :] = p_sel

        def comp(rs_chunks, n_chunks, col0):
            rws = pl.ds(rs_chunks * chunk, n_chunks * chunk)
            xb = x_ref[rws, :]
            rb = idx_ref[rws, 0:1]
            pb = psel_ref[rws, :]
            accu = jnp.zeros((n_chunks * chunk, h2), jnp.float32)
            for k in range(e_loc):
                w = ew_ref[k, :, col0:col0 + h2]
                ck = jnp.where(rb == my * e_loc + k, pb, 0.0)
                accu = accu + ck * jnp.dot(
                    xb, w, preferred_element_type=jnp.float32)
            partial_ref[rws, col0:col0 + h2] = accu

        kA0 = b4 * 4
        sA0 = (1 - b4) * 4
        kA1 = kA0 + b2 * 2
        sA1 = kA0 + (1 - b2) * 2
        sA2 = kA1 + (1 - b1)
        kB0a = b2 * 2
        kB0b = 4 + b2 * 2
        sB0a = (1 - b2) * 2
        sB0b = 4 + (1 - b2) * 2
        kB1a = kB0a + b1
        kB1b = kB0b + b1
        sB1a = kB0a + (1 - b1)
        sB1b = kB0b + (1 - b1)
        sB2 = (1 - b4) * 4 + b2 * 2 + b1

        comp(sA0, 4, 0)
        comp(sB0a, 2, h2)
        comp(sB0b, 2, h2)

        barrier_sem = pltpu.get_barrier_semaphore()
        for nbr in (nx, ny, nz):
            pl.semaphore_signal(barrier_sem, inc=1, device_id=(nbr,),
                                device_id_type=pl.DeviceIdType.MESH)
        pl.semaphore_wait(barrier_sem, 3)

        mk(partial_ref.at[rows(sA0, 4), A], rA0,
           A_s.at[0], A_r.at[0], nz).start()
        mk(partial_ref.at[rows(sB0a, 2), B], rB0.at[pl.ds(0, 2 * chunk)],
           B_s.at[0], B_r.at[0], ny).start()
        mk(partial_ref.at[rows(sB0b, 2), B], rB0.at[pl.ds(2 * chunk, 2 * chunk)],
           B_s.at[1], B_r.at[1], ny).start()

        comp(kA0, 4, 0)
        comp(kB0a, 2, h2)
        comp(kB0b, 2, h2)

        mk(rA0, rA0, A_s.at[0], A_r.at[0], nz).wait_recv()
        partial_ref[rows(sA1, 2), A] = (
            partial_ref[rows(sA1, 2), A]
            + rA0[pl.ds((1 - b2) * 2 * chunk, 2 * chunk), :])
        mk(partial_ref.at[rows(sA1, 2), A], rA1,
           A_s.at[1], A_r.at[1], ny).start()
        partial_ref[rows(kA1, 2), A] = (
            partial_ref[rows(kA1, 2), A]
            + rA0[pl.ds(b2 * 2 * chunk, 2 * chunk), :])
        mk(rB0.at[pl.ds(0, 2 * chunk)], rB0.at[pl.ds(0, 2 * chunk)],
           B_s.at[0], B_r.at[0], ny).wait_recv()
        mk(rB0.at[pl.ds(2 * chunk, 2 * chunk)], rB0.at[pl.ds(2 * chunk, 2 * chunk)],
           B_s.at[1], B_r.at[1], ny).wait_recv()
        partial_ref[rows(sB1a), B] = (
            partial_ref[rows(sB1a), B]
            + rB0[pl.ds((1 - b1) * chunk, chunk), :])
        partial_ref[rows(sB1b), B] = (
            partial_ref[rows(sB1b), B]
            + rB0[pl.ds(2 * chunk + (1 - b1) * chunk, chunk), :])
        mk(partial_ref.at[rows(sB1a), B], rB1.at[pl.ds(0, chunk)],
           B_s.at[2], B_r.at[2], nx).start()
        mk(partial_ref.at[rows(sB1b), B], rB1.at[pl.ds(chunk, chunk)],
           B_s.at[3], B_r.at[3], nx).start()
        partial_ref[rows(kB1a), B] = (
            partial_ref[rows(kB1a), B] + rB0[pl.ds(b1 * chunk, chunk), :])
        partial_ref[rows(kB1b), B] = (
            partial_ref[rows(kB1b), B]
            + rB0[pl.ds(2 * chunk + b1 * chunk, chunk), :])

        sh = jnp.dot(x_ref[rows(ell), :], sw_ref[:, :],
                     preferred_element_type=jnp.float32)

        mk(rA1, rA1, A_s.at[1], A_r.at[1], ny).wait_recv()
        partial_ref[rows(sA2), A] = (
            partial_ref[rows(sA2), A]
            + rA1[pl.ds((1 - b1) * chunk, chunk), :])
        mk(partial_ref.at[rows(sA2), A], rA2,
           A_s.at[2], A_r.at[2], nx).start()
        partial_ref[rows(ell), A] = (
            partial_ref[rows(ell), A] + rA1[pl.ds(b1 * chunk, chunk), :])
        mk(rB1.at[pl.ds(0, chunk)], rB1.at[pl.ds(0, chunk)],
           B_s.at[2], B_r.at[2], nx).wait_recv()
        mk(rB1.at[pl.ds(chunk, chunk)], rB1.at[pl.ds(chunk, chunk)],
           B_s.at[3], B_r.at[3], nx).wait_recv()
        partial_ref[rows(sB2), B] = (
            partial_ref[rows(sB2), B]
            + rB1[pl.ds((1 - b4) * chunk, chunk), :])
        mk(partial_ref.at[rows(sB2), B], rB2,
           B_s.at[4], B_r.at[4], nz).start()
        partial_ref[rows(ell), B] = (
            partial_ref[rows(ell), B] + rB1[pl.ds(b4 * chunk, chunk), :])

        mk(rA2, rA2, A_s.at[2], A_r.at[2], nx).wait_recv()
        mk(rB2, rB2, B_s.at[4], B_r.at[4], nz).wait_recv()
        out_ref[rows(ell), A] = (
            partial_ref[rows(ell), A] + rA2[:, :] + sh[:, 0:h2])
        out_ref[rows(ell), B] = (
            partial_ref[rows(ell), B] + rB2[:, :] + sh[:, h2:h])

        pairA = ell & ~1
        quadA = ell & ~3

        def oA(c, n=1):
            return out_ref.at[rows(c, n), A]

        def oB(c, n=1):
            return out_ref.at[rows(c, n), B]

        mk(oA(ell), oA(ell), aA_s.at[0], aA_r.at[0], nx).start()
        mk(oB(ell), oB(ell), aB_s.at[0], aB_r.at[0], nz).start()

        mk(oA(ell ^ 1), oA(ell ^ 1), aA_s.at[0], aA_r.at[0], nx).wait_recv()
        mk(oB(ell ^ 4), oB(ell ^ 4), aB_s.at[0], aB_r.at[0], nz).wait_recv()
        mk(oA(pairA, 2), oA(pairA, 2), aA_s.at[1], aA_r.at[1], ny).start()
        mk(oB(ell), oB(ell), aB_s.at[1], aB_r.at[1], nx).start()
        mk(oB(ell ^ 4), oB(ell ^ 4), aB_s.at[2], aB_r.at[2], nx).start()

        mk(oA(pairA ^ 2, 2), oA(pairA ^ 2, 2),
           aA_s.at[1], aA_r.at[1], ny).wait_recv()
        mk(oB(ell ^ 1), oB(ell ^ 1), aB_s.at[1], aB_r.at[1], nx).wait_recv()
        mk(oB(ell ^ 1 ^ 4), oB(ell ^ 1 ^ 4),
           aB_s.at[2], aB_r.at[2], nx).wait_recv()
        mk(oA(quadA, 4), oA(quadA, 4), aA_s.at[2], aA_r.at[2], nz).start()
        mk(oB(pairA, 2), oB(pairA, 2), aB_s.at[3], aB_r.at[3], ny).start()
        mk(oB(pairA ^ 4, 2), oB(pairA ^ 4, 2),
           aB_s.at[4], aB_r.at[4], ny).start()

        mk(oA(quadA ^ 4, 4), oA(quadA ^ 4, 4),
           aA_s.at[2], aA_r.at[2], nz).wait_recv()
        mk(oB(pairA ^ 2, 2), oB(pairA ^ 2, 2),
           aB_s.at[3], aB_r.at[3], ny).wait_recv()
        mk(oB(pairA ^ 2 ^ 4, 2), oB(pairA ^ 2 ^ 4, 2),
           aB_s.at[4], aB_r.at[4], ny).wait_recv()

        mk(rA0, rA0, A_s.at[0], A_r.at[0], nz).wait_send()
        mk(rA1, rA1, A_s.at[1], A_r.at[1], ny).wait_send()
        mk(rA2, rA2, A_s.at[2], A_r.at[2], nx).wait_send()
        mk(rB0.at[pl.ds(0, 2 * chunk)], rB0.at[pl.ds(0, 2 * chunk)],
           B_s.at[0], B_r.at[0], ny).wait_send()
        mk(rB0.at[pl.ds(0, 2 * chunk)], rB0.at[pl.ds(0, 2 * chunk)],
           B_s.at[1], B_r.at[1], ny).wait_send()
        mk(rB2, rB2, B_s.at[2], B_r.at[2], nx).wait_send()
        mk(rB2, rB2, B_s.at[3], B_r.at[3], nx).wait_send()
        mk(rB2, rB2, B_s.at[4], B_r.at[4], nz).wait_send()
        mk(oA(ell), oA(ell), aA_s.at[0], aA_r.at[0], nx).wait_send()
        mk(oA(pairA, 2), oA(pairA, 2), aA_s.at[1], aA_r.at[1], ny).wait_send()
        mk(oA(quadA, 4), oA(quadA, 4), aA_s.at[2], aA_r.at[2], nz).wait_send()
        mk(oB(ell), oB(ell), aB_s.at[0], aB_r.at[0], nz).wait_send()
        mk(oB(ell), oB(ell), aB_s.at[1], aB_r.at[1], nx).wait_send()
        mk(oB(ell), oB(ell), aB_s.at[2], aB_r.at[2], nx).wait_send()
        mk(oB(pairA, 2), oB(pairA, 2), aB_s.at[3], aB_r.at[3], ny).wait_send()
        mk(oB(pairA, 2), oB(pairA, 2), aB_s.at[4], aB_r.at[4], ny).wait_send()

    return pl.pallas_call(
        body,
        out_shape=jax.ShapeDtypeStruct((n_tok, h), jnp.float32),
        in_specs=[pl.BlockSpec(memory_space=pltpu.VMEM)] * 5,
        out_specs=pl.BlockSpec(memory_space=pltpu.VMEM),
        scratch_shapes=[
            pltpu.VMEM((n_tok, h), jnp.float32),
            pltpu.VMEM((4 * chunk, h2), jnp.float32),
            pltpu.VMEM((2 * chunk, h2), jnp.float32),
            pltpu.VMEM((chunk, h2), jnp.float32),
            pltpu.VMEM((4 * chunk, h2), jnp.float32),
            pltpu.VMEM((2 * chunk, h2), jnp.float32),
            pltpu.VMEM((chunk, h2), jnp.float32),
            pltpu.VMEM((n_tok, 1), jnp.float32),
            pltpu.SemaphoreType.DMA((3,)),
            pltpu.SemaphoreType.DMA((3,)),
            pltpu.SemaphoreType.DMA((5,)),
            pltpu.SemaphoreType.DMA((5,)),
            pltpu.SemaphoreType.DMA((3,)),
            pltpu.SemaphoreType.DMA((3,)),
            pltpu.SemaphoreType.DMA((5,)),
            pltpu.SemaphoreType.DMA((5,)),
        ],
        compiler_params=pltpu.CompilerParams(collective_id=0),
    )(x, router_W, route_idx, expert_W, shared_W)


# device time: 58493 ns/iter; 1.1546x vs baseline; 1.1546x over previous
import jax
import jax.numpy as jnp
from jax import lax
from jax.experimental import pallas as pl
from jax.experimental.pallas import tpu as pltpu

N_DEV = 8


def kernel(x, router_W, route_idx, expert_W, shared_W):
    n_tok, d = x.shape
    n_exp = router_W.shape[1]
    e_loc, _, h = expert_W.shape
    chunk = n_tok // N_DEV
    wA = 384
    wB = 384
    wC = h - wA - wB

    def body(x_ref, rw_ref, idx_ref, ew_ref, sw_ref, out_ref,
             partial_ref, rA0, rA1, rA2, rB0, rB1, rB2, rC0, rC1, rC2,
             psel_ref,
             A_s, A_r, B_s, B_r, C_s, C_r, aA_s, aA_r, aB_s, aB_r,
             aC_s, aC_r):
        my = lax.axis_index("i")
        ell = my ^ ((my >> 1) & 1)
        b1 = ell & 1
        b2 = (ell >> 1) & 1
        b4 = (ell >> 2) & 1

        def logi(l):
            return l ^ ((l >> 1) & 1)

        nx = logi(ell ^ 1)
        ny = logi(ell ^ 2)
        nz = logi(ell ^ 4)

        def rows(c, n=1):
            return pl.ds(c * chunk, n * chunk)

        A = pl.ds(0, wA)
        B = pl.ds(wA, wB)
        C = pl.ds(wA + wB, wC)

        def mk(src, dst, ssem, rsem, dev):
            return pltpu.make_async_remote_copy(
                src_ref=src, dst_ref=dst, send_sem=ssem, recv_sem=rsem,
                device_id=(dev,), device_id_type=pl.DeviceIdType.MESH)

        xv = x_ref[:, :]

        scores = jnp.dot(xv, rw_ref[:, :], preferred_element_type=jnp.float32)
        m = jnp.max(scores, axis=-1, keepdims=True)
        p = jnp.exp(scores - m)
        probs = p / jnp.sum(p, axis=-1, keepdims=True)
        ridx = idx_ref[:, 0:1]
        e_ids = lax.broadcasted_iota(jnp.int32, (n_tok, n_exp), 1)
        p_sel = jnp.sum(jnp.where(e_ids == ridx, probs, 0.0),
                        axis=1, keepdims=True)
        psel_ref[:, :] = p_sel

        def comp(rs_chunks, n_chunks, col0, w):
            rws = pl.ds(rs_chunks * chunk, n_chunks * chunk)
            xb = x_ref[rws, :]
            rb = idx_ref[rws, 0:1]
            pb = psel_ref[rws, :]
            accu = jnp.zeros((n_chunks * chunk, w), jnp.float32)
            for k in range(e_loc):
                wk = ew_ref[k, :, col0:col0 + w]
                ck = jnp.where(rb == my * e_loc + k, pb, 0.0)
                accu = accu + ck * jnp.dot(
                    xb, wk, preferred_element_type=jnp.float32)
            partial_ref[rws, col0:col0 + w] = accu

        kA0 = b4 * 4
        sA0 = (1 - b4) * 4
        kA1 = kA0 + b2 * 2
        sA1 = kA0 + (1 - b2) * 2
        sA2 = kA1 + (1 - b1)
        kB0a = b2 * 2
        kB0b = 4 + b2 * 2
        sB0a = (1 - b2) * 2
        sB0b = 4 + (1 - b2) * 2
        kB1a = kB0a + b1
        kB1b = kB0b + b1
        sB1a = kB0a + (1 - b1)
        sB1b = kB0b + (1 - b1)
        sB2 = (1 - b4) * 4 + b2 * 2 + b1
        sC2 = b1 + 4 * b4 + 2 * (1 - b2)

        comp(sA0, 4, 0, wA)
        comp(sB0a, 2, wA, wB)
        comp(sB0b, 2, wA, wB)
        for i in range(4):
            comp((1 - b1) + 2 * i, 1, wA + wB, wC)

        barrier_sem = pltpu.get_barrier_semaphore()
        for nbr in (nx, ny, nz):
            pl.semaphore_signal(barrier_sem, inc=1, device_id=(nbr,),
                                device_id_type=pl.DeviceIdType.MESH)
        pl.semaphore_wait(barrier_sem, 3)

        mk(partial_ref.at[rows(sA0, 4), A], rA0,
           A_s.at[0], A_r.at[0], nz).start()
        mk(partial_ref.at[rows(sB0a, 2), B], rB0.at[pl.ds(0, 2 * chunk)],
           B_s.at[0], B_r.at[0], ny).start()
        mk(partial_ref.at[rows(sB0b, 2), B], rB0.at[pl.ds(2 * chunk, 2 * chunk)],
           B_s.at[1], B_r.at[1], ny).start()
        for i in range(4):
            mk(partial_ref.at[rows((1 - b1) + 2 * i), C],
               rC0.at[pl.ds(i * chunk, chunk)],
               C_s.at[i], C_r.at[i], nx).start()

        comp(kA0, 4, 0, wA)
        comp(kB0a, 2, wA, wB)
        comp(kB0b, 2, wA, wB)
        for i in range(4):
            comp(b1 + 2 * i, 1, wA + wB, wC)

        mk(rA0, rA0, A_s.at[0], A_r.at[0], nz).wait_recv()
        partial_ref[rows(sA1, 2), A] = (
            partial_ref[rows(sA1, 2), A]
            + rA0[pl.ds((1 - b2) * 2 * chunk, 2 * chunk), :])
        mk(partial_ref.at[rows(sA1, 2), A], rA1,
           A_s.at[1], A_r.at[1], ny).start()
        partial_ref[rows(kA1, 2), A] = (
            partial_ref[rows(kA1, 2), A]
            + rA0[pl.ds(b2 * 2 * chunk, 2 * chunk), :])

        mk(rB0.at[pl.ds(0, 2 * chunk)], rB0.at[pl.ds(0, 2 * chunk)],
           B_s.at[0], B_r.at[0], ny).wait_recv()
        mk(rB0.at[pl.ds(2 * chunk, 2 * chunk)], rB0.at[pl.ds(2 * chunk, 2 * chunk)],
           B_s.at[1], B_r.at[1], ny).wait_recv()
        partial_ref[rows(sB1a), B] = (
            partial_ref[rows(sB1a), B]
            + rB0[pl.ds((1 - b1) * chunk, chunk), :])
        partial_ref[rows(sB1b), B] = (
            partial_ref[rows(sB1b), B]
            + rB0[pl.ds(2 * chunk + (1 - b1) * chunk, chunk), :])
        mk(partial_ref.at[rows(sB1a), B], rB1.at[pl.ds(0, chunk)],
           B_s.at[2], B_r.at[2], nx).start()
        mk(partial_ref.at[rows(sB1b), B], rB1.at[pl.ds(chunk, chunk)],
           B_s.at[3], B_r.at[3], nx).start()
        partial_ref[rows(kB1a), B] = (
            partial_ref[rows(kB1a), B] + rB0[pl.ds(b1 * chunk, chunk), :])
        partial_ref[rows(kB1b), B] = (
            partial_ref[rows(kB1b), B]
            + rB0[pl.ds(2 * chunk + b1 * chunk, chunk), :])

        for i in range(4):
            mk(rC0.at[pl.ds(i * chunk, chunk)], rC0.at[pl.ds(i * chunk, chunk)],
               C_s.at[i], C_r.at[i], nx).wait_recv()
        for j in range(2):
            i = (1 - b4) * 2 + j
            cc = b1 + 2 * i
            partial_ref[rows(cc), C] = (
                partial_ref[rows(cc), C] + rC0[pl.ds(i * chunk, chunk), :])
            mk(partial_ref.at[rows(cc), C], rC1.at[pl.ds(j * chunk, chunk)],
               C_s.at[4 + j], C_r.at[4 + j], nz).start()
        for j in range(2):
            i = b4 * 2 + j
            cc = b1 + 2 * i
            partial_ref[rows(cc), C] = (
                partial_ref[rows(cc), C] + rC0[pl.ds(i * chunk, chunk), :])

        sh = jnp.dot(x_ref[rows(ell), :], sw_ref[:, :],
                     preferred_element_type=jnp.float32)

        mk(rA1, rA1, A_s.at[1], A_r.at[1], ny).wait_recv()
        partial_ref[rows(sA2), A] = (
            partial_ref[rows(sA2), A]
            + rA1[pl.ds((1 - b1) * chunk, chunk), :])
        mk(partial_ref.at[rows(sA2), A], rA2,
           A_s.at[2], A_r.at[2], nx).start()
        partial_ref[rows(ell), A] = (
            partial_ref[rows(ell), A] + rA1[pl.ds(b1 * chunk, chunk), :])

        mk(rB1.at[pl.ds(0, chunk)], rB1.at[pl.ds(0, chunk)],
           B_s.at[2], B_r.at[2], nx).wait_recv()
        mk(rB1.at[pl.ds(chunk, chunk)], rB1.at[pl.ds(chunk, chunk)],
           B_s.at[3], B_r.at[3], nx).wait_recv()
        partial_ref[rows(sB2), B] = (
            partial_ref[rows(sB2), B]
            + rB1[pl.ds((1 - b4) * chunk, chunk), :])
        mk(partial_ref.at[rows(sB2), B], rB2,
           B_s.at[4], B_r.at[4], nz).start()
        partial_ref[rows(ell), B] = (
            partial_ref[rows(ell), B] + rB1[pl.ds(b4 * chunk, chunk), :])

        mk(rC1.at[pl.ds(0, chunk)], rC1.at[pl.ds(0, chunk)],
           C_s.at[4], C_r.at[4], nz).wait_recv()
        mk(rC1.at[pl.ds(chunk, chunk)], rC1.at[pl.ds(chunk, chunk)],
           C_s.at[5], C_r.at[5], nz).wait_recv()
        partial_ref[rows(sC2), C] = (
            partial_ref[rows(sC2), C]
            + rC1[pl.ds((1 - b2) * chunk, chunk), :])
        mk(partial_ref.at[rows(sC2), C], rC2,
           C_s.at[6], C_r.at[6], ny).start()
        partial_ref[rows(ell), C] = (
            partial_ref[rows(ell), C] + rC1[pl.ds(b2 * chunk, chunk), :])

        mk(rA2, rA2, A_s.at[2], A_r.at[2], nx).wait_recv()
        mk(rB2, rB2, B_s.at[4], B_r.at[4], nz).wait_recv()
        mk(rC2, rC2, C_s.at[6], C_r.at[6], ny).wait_recv()
        out_ref[rows(ell), A] = (
            partial_ref[rows(ell), A] + rA2[:, :] + sh[:, 0:wA])
        out_ref[rows(ell), B] = (
            partial_ref[rows(ell), B] + rB2[:, :] + sh[:, wA:wA + wB])
        out_ref[rows(ell), C] = (
            partial_ref[rows(ell), C] + rC2[:, :] + sh[:, wA + wB:h])

        pairA = ell & ~1
        quadA = ell & ~3

        def oA(c, n=1):
            return out_ref.at[rows(c, n), A]

        def oB(c, n=1):
            return out_ref.at[rows(c, n), B]

        def oC(c, n=1):
            return out_ref.at[rows(c, n), C]

        mk(oA(ell), oA(ell), aA_s.at[0], aA_r.at[0], nx).start()
        mk(oB(ell), oB(ell), aB_s.at[0], aB_r.at[0], nz).start()
        mk(oC(ell), oC(ell), aC_s.at[0], aC_r.at[0], ny).start()

        mk(oA(ell ^ 1), oA(ell ^ 1), aA_s.at[0], aA_r.at[0], nx).wait_recv()
        mk(oB(ell ^ 4), oB(ell ^ 4), aB_s.at[0], aB_r.at[0], nz).wait_recv()
        mk(oC(ell ^ 2), oC(ell ^ 2), aC_s.at[0], aC_r.at[0], ny).wait_recv()
        mk(oA(pairA, 2), oA(pairA, 2), aA_s.at[1], aA_r.at[1], ny).start()
        mk(oB(ell), oB(ell), aB_s.at[1], aB_r.at[1], nx).start()
        mk(oB(ell ^ 4), oB(ell ^ 4), aB_s.at[2], aB_r.at[2], nx).start()
        mk(oC(ell), oC(ell), aC_s.at[1], aC_r.at[1], nz).start()
        mk(oC(ell ^ 2), oC(ell ^ 2), aC_s.at[2], aC_r.at[2], nz).start()

        mk(oA(pairA ^ 2, 2), oA(pairA ^ 2, 2),
           aA_s.at[1], aA_r.at[1], ny).wait_recv()
        mk(oB(ell ^ 1), oB(ell ^ 1), aB_s.at[1], aB_r.at[1], nx).wait_recv()
        mk(oB(ell ^ 1 ^ 4), oB(ell ^ 1 ^ 4),
           aB_s.at[2], aB_r.at[2], nx).wait_recv()
        mk(oC(ell ^ 4), oC(ell ^ 4), aC_s.at[1], aC_r.at[1], nz).wait_recv()
        mk(oC(ell ^ 4 ^ 2), oC(ell ^ 4 ^ 2),
           aC_s.at[2], aC_r.at[2], nz).wait_recv()
        mk(oA(quadA, 4), oA(quadA, 4), aA_s.at[2], aA_r.at[2], nz).start()
        mk(oB(pairA, 2), oB(pairA, 2), aB_s.at[3], aB_r.at[3], ny).start()
        mk(oB(pairA ^ 4, 2), oB(pairA ^ 4, 2),
           aB_s.at[4], aB_r.at[4], ny).start()
        mk(oC(ell), oC(ell), aC_s.at[3], aC_r.at[3], nx).start()
        mk(oC(ell ^ 2), oC(ell ^ 2), aC_s.at[4], aC_r.at[4], nx).start()
        mk(oC(ell ^ 4), oC(ell ^ 4), aC_s.at[5], aC_r.at[5], nx).start()
        mk(oC(ell ^ 6), oC(ell ^ 6), aC_s.at[6], aC_r.at[6], nx).start()

        mk(oA(quadA ^ 4, 4), oA(quadA ^ 4, 4),
           aA_s.at[2], aA_r.at[2], nz).wait_recv()
        mk(oB(pairA ^ 2, 2), oB(pairA ^ 2, 2),
           aB_s.at[3], aB_r.at[3], ny).wait_recv()
        mk(oB(pairA ^ 2 ^ 4, 2), oB(pairA ^ 2 ^ 4, 2),
           aB_s.at[4], aB_r.at[4], ny).wait_recv()
        mk(oC(ell ^ 1), oC(ell ^ 1), aC_s.at[3], aC_r.at[3], nx).wait_recv()
        mk(oC(ell ^ 3), oC(ell ^ 3), aC_s.at[4], aC_r.at[4], nx).wait_recv()
        mk(oC(ell ^ 5), oC(ell ^ 5), aC_s.at[5], aC_r.at[5], nx).wait_recv()
        mk(oC(ell ^ 7), oC(ell ^ 7), aC_s.at[6], aC_r.at[6], nx).wait_recv()

        mk(rA0, rA0, A_s.at[0], A_r.at[0], nz).wait_send()
        mk(rA1, rA1, A_s.at[1], A_r.at[1], ny).wait_send()
        mk(rA2, rA2, A_s.at[2], A_r.at[2], nx).wait_send()
        mk(rB0.at[pl.ds(0, 2 * chunk)], rB0.at[pl.ds(0, 2 * chunk)],
           B_s.at[0], B_r.at[0], ny).wait_send()
        mk(rB0.at[pl.ds(0, 2 * chunk)], rB0.at[pl.ds(0, 2 * chunk)],
           B_s.at[1], B_r.at[1], ny).wait_send()
        mk(rB2, rB2, B_s.at[2], B_r.at[2], nx).wait_send()
        mk(rB2, rB2, B_s.at[3], B_r.at[3], nx).wait_send()
        mk(rB2, rB2, B_s.at[4], B_r.at[4], nz).wait_send()
        for i in range(7):
            mk(rC2, rC2, C_s.at[i], C_r.at[i], nx).wait_send()
        mk(oA(ell), oA(ell), aA_s.at[0], aA_r.at[0], nx).wait_send()
        mk(oA(pairA, 2), oA(pairA, 2), aA_s.at[1], aA_r.at[1], ny).wait_send()
        mk(oA(quadA, 4), oA(quadA, 4), aA_s.at[2], aA_r.at[2], nz).wait_send()
        mk(oB(ell), oB(ell), aB_s.at[0], aB_r.at[0], nz).wait_send()
        mk(oB(ell), oB(ell), aB_s.at[1], aB_r.at[1], nx).wait_send()
        mk(oB(ell), oB(ell), aB_s.at[2], aB_r.at[2], nx).wait_send()
        mk(oB(pairA, 2), oB(pairA, 2), aB_s.at[3], aB_r.at[3], ny).wait_send()
        mk(oB(pairA, 2), oB(pairA, 2), aB_s.at[4], aB_r.at[4], ny).wait_send()
        for i in range(7):
            mk(oC(ell), oC(ell), aC_s.at[i], aC_r.at[i], ny).wait_send()

    return pl.pallas_call(
        body,
        out_shape=jax.ShapeDtypeStruct((n_tok, h), jnp.float32),
        in_specs=[pl.BlockSpec(memory_space=pltpu.VMEM)] * 5,
        out_specs=pl.BlockSpec(memory_space=pltpu.VMEM),
        scratch_shapes=[
            pltpu.VMEM((n_tok, h), jnp.float32),
            pltpu.VMEM((4 * chunk, wA), jnp.float32),
            pltpu.VMEM((2 * chunk, wA), jnp.float32),
            pltpu.VMEM((chunk, wA), jnp.float32),
            pltpu.VMEM((4 * chunk, wB), jnp.float32),
            pltpu.VMEM((2 * chunk, wB), jnp.float32),
            pltpu.VMEM((chunk, wB), jnp.float32),
            pltpu.VMEM((4 * chunk, wC), jnp.float32),
            pltpu.VMEM((2 * chunk, wC), jnp.float32),
            pltpu.VMEM((chunk, wC), jnp.float32),
            pltpu.VMEM((n_tok, 1), jnp.float32),
            pltpu.SemaphoreType.DMA((3,)),
            pltpu.SemaphoreType.DMA((3,)),
            pltpu.SemaphoreType.DMA((5,)),
            pltpu.SemaphoreType.DMA((5,)),
            pltpu.SemaphoreType.DMA((7,)),
            pltpu.SemaphoreType.DMA((7,)),
            pltpu.SemaphoreType.DMA((3,)),
            pltpu.SemaphoreType.DMA((3,)),
            pltpu.SemaphoreType.DMA((5,)),
            pltpu.SemaphoreType.DMA((5,)),
            pltpu.SemaphoreType.DMA((7,)),
            pltpu.SemaphoreType.DMA((7,)),
        ],
        compiler_params=pltpu.CompilerParams(collective_id=0),
    )(x, router_W, route_idx, expert_W, shared_W)


# device time: 44094 ns/iter; 1.5316x vs baseline; 1.3266x over previous
import jax
import jax.numpy as jnp
from jax import lax
from jax.experimental import pallas as pl
from jax.experimental.pallas import tpu as pltpu

N_DEV = 8


def kernel(x, router_W, route_idx, expert_W, shared_W):
    n_tok, d = x.shape
    n_exp = router_W.shape[1]
    e_loc, _, h = expert_W.shape
    chunk = n_tok // N_DEV
    wA = 384
    wB = 384
    wC = h - wA - wB

    def body(x_ref, rw_ref, idx_ref, ew_ref, sw_ref, out_ref,
             partial_ref, gbuf, x16, rA0, rA1, rA2, rB0, rB1, rB2,
             rC0, rC1, rC2, psel_ref,
             A_s, A_r, B_s, B_r, C_s, C_r, aA_s, aA_r, aB_s, aB_r,
             aC_s, aC_r):
        my = lax.axis_index("i")
        ell = my ^ ((my >> 1) & 1)
        b1 = ell & 1
        b2 = (ell >> 1) & 1
        b4 = (ell >> 2) & 1

        def logi(l):
            return l ^ ((l >> 1) & 1)

        nx = logi(ell ^ 1)
        ny = logi(ell ^ 2)
        nz = logi(ell ^ 4)

        def rows(c, n=1):
            return pl.ds(c * chunk, n * chunk)

        A = pl.ds(0, wA)
        B = pl.ds(wA, wB)
        C = pl.ds(wA + wB, wC)

        def mk(src, dst, ssem, rsem, dev):
            return pltpu.make_async_remote_copy(
                src_ref=src, dst_ref=dst, send_sem=ssem, recv_sem=rsem,
                device_id=(dev,), device_id_type=pl.DeviceIdType.MESH)

        xv = x_ref[:, :]
        x16[:, :] = xv.astype(jnp.bfloat16)

        scores = jnp.dot(xv, rw_ref[:, :], preferred_element_type=jnp.float32)
        m = jnp.max(scores, axis=-1, keepdims=True)
        p = jnp.exp(scores - m)
        probs = p / jnp.sum(p, axis=-1, keepdims=True)
        ridx = idx_ref[:, 0:1]
        e_ids = lax.broadcasted_iota(jnp.int32, (n_tok, n_exp), 1)
        p_sel = jnp.sum(jnp.where(e_ids == ridx, probs, 0.0),
                        axis=1, keepdims=True)
        psel_ref[:, :] = p_sel

        def comp(rs_chunks, n_chunks, col0, w):
            rws = pl.ds(rs_chunks * chunk, n_chunks * chunk)
            xb = x16[rws, :]
            rb = idx_ref[rws, 0:1]
            pb = psel_ref[rws, :]
            accu = jnp.zeros((n_chunks * chunk, w), jnp.float32)
            for k in range(e_loc):
                wk = ew_ref[k, :, col0:col0 + w].astype(jnp.bfloat16)
                ck = jnp.where(rb == my * e_loc + k, pb, 0.0)
                accu = accu + ck * jnp.dot(
                    xb, wk, preferred_element_type=jnp.float32)
            partial_ref[rws, col0:col0 + w] = accu.astype(jnp.bfloat16)

        kA0 = b4 * 4
        sA0 = (1 - b4) * 4
        kA1 = kA0 + b2 * 2
        sA1 = kA0 + (1 - b2) * 2
        sA2 = kA1 + (1 - b1)
        kB0a = b2 * 2
        kB0b = 4 + b2 * 2
        sB0a = (1 - b2) * 2
        sB0b = 4 + (1 - b2) * 2
        kB1a = kB0a + b1
        kB1b = kB0b + b1
        sB1a = kB0a + (1 - b1)
        sB1b = kB0b + (1 - b1)
        sB2 = (1 - b4) * 4 + b2 * 2 + b1
        sC2 = b1 + 4 * b4 + 2 * (1 - b2)

        comp(sA0, 4, 0, wA)
        comp(sB0a, 2, wA, wB)
        comp(sB0b, 2, wA, wB)
        for i in range(4):
            comp((1 - b1) + 2 * i, 1, wA + wB, wC)

        barrier_sem = pltpu.get_barrier_semaphore()
        for nbr in (nx, ny, nz):
            pl.semaphore_signal(barrier_sem, inc=1, device_id=(nbr,),
                                device_id_type=pl.DeviceIdType.MESH)
        pl.semaphore_wait(barrier_sem, 3)

        mk(partial_ref.at[rows(sA0, 4), A], rA0,
           A_s.at[0], A_r.at[0], nz).start()
        mk(partial_ref.at[rows(sB0a, 2), B], rB0.at[pl.ds(0, 2 * chunk)],
           B_s.at[0], B_r.at[0], ny).start()
        mk(partial_ref.at[rows(sB0b, 2), B], rB0.at[pl.ds(2 * chunk, 2 * chunk)],
           B_s.at[1], B_r.at[1], ny).start()
        for i in range(4):
            mk(partial_ref.at[rows((1 - b1) + 2 * i), C],
               rC0.at[pl.ds(i * chunk, chunk)],
               C_s.at[i], C_r.at[i], nx).start()

        comp(kA0, 4, 0, wA)
        comp(kB0a, 2, wA, wB)
        comp(kB0b, 2, wA, wB)
        for i in range(4):
            comp(b1 + 2 * i, 1, wA + wB, wC)

        mk(rA0, rA0, A_s.at[0], A_r.at[0], nz).wait_recv()
        partial_ref[rows(sA1, 2), A] = (
            partial_ref[rows(sA1, 2), A]
            + rA0[pl.ds((1 - b2) * 2 * chunk, 2 * chunk), :])
        mk(partial_ref.at[rows(sA1, 2), A], rA1,
           A_s.at[1], A_r.at[1], ny).start()
        partial_ref[rows(kA1, 2), A] = (
            partial_ref[rows(kA1, 2), A]
            + rA0[pl.ds(b2 * 2 * chunk, 2 * chunk), :])

        mk(rB0.at[pl.ds(0, 2 * chunk)], rB0.at[pl.ds(0, 2 * chunk)],
           B_s.at[0], B_r.at[0], ny).wait_recv()
        mk(rB0.at[pl.ds(2 * chunk, 2 * chunk)], rB0.at[pl.ds(2 * chunk, 2 * chunk)],
           B_s.at[1], B_r.at[1], ny).wait_recv()
        partial_ref[rows(sB1a), B] = (
            partial_ref[rows(sB1a), B]
            + rB0[pl.ds((1 - b1) * chunk, chunk), :])
        partial_ref[rows(sB1b), B] = (
            partial_ref[rows(sB1b), B]
            + rB0[pl.ds(2 * chunk + (1 - b1) * chunk, chunk), :])
        mk(partial_ref.at[rows(sB1a), B], rB1.at[pl.ds(0, chunk)],
           B_s.at[2], B_r.at[2], nx).start()
        mk(partial_ref.at[rows(sB1b), B], rB1.at[pl.ds(chunk, chunk)],
           B_s.at[3], B_r.at[3], nx).start()
        partial_ref[rows(kB1a), B] = (
            partial_ref[rows(kB1a), B] + rB0[pl.ds(b1 * chunk, chunk), :])
        partial_ref[rows(kB1b), B] = (
            partial_ref[rows(kB1b), B]
            + rB0[pl.ds(2 * chunk + b1 * chunk, chunk), :])

        for i in range(4):
            mk(rC0.at[pl.ds(i * chunk, chunk)], rC0.at[pl.ds(i * chunk, chunk)],
               C_s.at[i], C_r.at[i], nx).wait_recv()
        for j in range(2):
            i = (1 - b4) * 2 + j
            cc = b1 + 2 * i
            partial_ref[rows(cc), C] = (
                partial_ref[rows(cc), C] + rC0[pl.ds(i * chunk, chunk), :])
            mk(partial_ref.at[rows(cc), C], rC1.at[pl.ds(j * chunk, chunk)],
               C_s.at[4 + j], C_r.at[4 + j], nz).start()
        for j in range(2):
            i = b4 * 2 + j
            cc = b1 + 2 * i
            partial_ref[rows(cc), C] = (
                partial_ref[rows(cc), C] + rC0[pl.ds(i * chunk, chunk), :])

        sh = jnp.dot(x_ref[rows(ell), :], sw_ref[:, :],
                     preferred_element_type=jnp.float32)

        mk(rA1, rA1, A_s.at[1], A_r.at[1], ny).wait_recv()
        partial_ref[rows(sA2), A] = (
            partial_ref[rows(sA2), A]
            + rA1[pl.ds((1 - b1) * chunk, chunk), :])
        mk(partial_ref.at[rows(sA2), A], rA2,
           A_s.at[2], A_r.at[2], nx).start()
        partial_ref[rows(ell), A] = (
            partial_ref[rows(ell), A] + rA1[pl.ds(b1 * chunk, chunk), :])

        mk(rB1.at[pl.ds(0, chunk)], rB1.at[pl.ds(0, chunk)],
           B_s.at[2], B_r.at[2], nx).wait_recv()
        mk(rB1.at[pl.ds(chunk, chunk)], rB1.at[pl.ds(chunk, chunk)],
           B_s.at[3], B_r.at[3], nx).wait_recv()
        partial_ref[rows(sB2), B] = (
            partial_ref[rows(sB2), B]
            + rB1[pl.ds((1 - b4) * chunk, chunk), :])
        mk(partial_ref.at[rows(sB2), B], rB2,
           B_s.at[4], B_r.at[4], nz).start()
        partial_ref[rows(ell), B] = (
            partial_ref[rows(ell), B] + rB1[pl.ds(b4 * chunk, chunk), :])

        mk(rC1.at[pl.ds(0, chunk)], rC1.at[pl.ds(0, chunk)],
           C_s.at[4], C_r.at[4], nz).wait_recv()
        mk(rC1.at[pl.ds(chunk, chunk)], rC1.at[pl.ds(chunk, chunk)],
           C_s.at[5], C_r.at[5], nz).wait_recv()
        partial_ref[rows(sC2), C] = (
            partial_ref[rows(sC2), C]
            + rC1[pl.ds((1 - b2) * chunk, chunk), :])
        mk(partial_ref.at[rows(sC2), C], rC2,
           C_s.at[6], C_r.at[6], ny).start()
        partial_ref[rows(ell), C] = (
            partial_ref[rows(ell), C] + rC1[pl.ds(b2 * chunk, chunk), :])

        mk(rA2, rA2, A_s.at[2], A_r.at[2], nx).wait_recv()
        mk(rB2, rB2, B_s.at[4], B_r.at[4], nz).wait_recv()
        mk(rC2, rC2, C_s.at[6], C_r.at[6], ny).wait_recv()
        vA = (partial_ref[rows(ell), A] + rA2[:, :]).astype(jnp.float32) \
            + sh[:, 0:wA]
        vB = (partial_ref[rows(ell), B] + rB2[:, :]).astype(jnp.float32) \
            + sh[:, wA:wA + wB]
        vC = (partial_ref[rows(ell), C] + rC2[:, :]).astype(jnp.float32) \
            + sh[:, wA + wB:h]
        gbuf[rows(ell), A] = vA.astype(jnp.bfloat16)
        gbuf[rows(ell), B] = vB.astype(jnp.bfloat16)
        gbuf[rows(ell), C] = vC.astype(jnp.bfloat16)

        pairA = ell & ~1
        quadA = ell & ~3

        def gA(c, n=1):
            return gbuf.at[rows(c, n), A]

        def gB(c, n=1):
            return gbuf.at[rows(c, n), B]

        def gC(c, n=1):
            return gbuf.at[rows(c, n), C]

        mk(gA(ell), gA(ell), aA_s.at[0], aA_r.at[0], nx).start()
        mk(gB(ell), gB(ell), aB_s.at[0], aB_r.at[0], nz).start()
        mk(gC(ell), gC(ell), aC_s.at[0], aC_r.at[0], ny).start()
        out_ref[rows(ell), A] = vA
        out_ref[rows(ell), B] = vB
        out_ref[rows(ell), C] = vC

        mk(gA(ell ^ 1), gA(ell ^ 1), aA_s.at[0], aA_r.at[0], nx).wait_recv()
        mk(gB(ell ^ 4), gB(ell ^ 4), aB_s.at[0], aB_r.at[0], nz).wait_recv()
        mk(gC(ell ^ 2), gC(ell ^ 2), aC_s.at[0], aC_r.at[0], ny).wait_recv()
        mk(gA(pairA, 2), gA(pairA, 2), aA_s.at[1], aA_r.at[1], ny).start()
        mk(gB(ell), gB(ell), aB_s.at[1], aB_r.at[1], nx).start()
        mk(gB(ell ^ 4), gB(ell ^ 4), aB_s.at[2], aB_r.at[2], nx).start()
        mk(gC(ell), gC(ell), aC_s.at[1], aC_r.at[1], nz).start()
        mk(gC(ell ^ 2), gC(ell ^ 2), aC_s.at[2], aC_r.at[2], nz).start()
        out_ref[rows(ell ^ 1), A] = gbuf[rows(ell ^ 1), A].astype(jnp.float32)
        out_ref[rows(ell ^ 4), B] = gbuf[rows(ell ^ 4), B].astype(jnp.float32)
        out_ref[rows(ell ^ 2), C] = gbuf[rows(ell ^ 2), C].astype(jnp.float32)

        mk(gA(pairA ^ 2, 2), gA(pairA ^ 2, 2),
           aA_s.at[1], aA_r.at[1], ny).wait_recv()
        mk(gB(ell ^ 1), gB(ell ^ 1), aB_s.at[1], aB_r.at[1], nx).wait_recv()
        mk(gB(ell ^ 1 ^ 4), gB(ell ^ 1 ^ 4),
           aB_s.at[2], aB_r.at[2], nx).wait_recv()
        mk(gC(ell ^ 4), gC(ell ^ 4), aC_s.at[1], aC_r.at[1], nz).wait_recv()
        mk(gC(ell ^ 4 ^ 2), gC(ell ^ 4 ^ 2),
           aC_s.at[2], aC_r.at[2], nz).wait_recv()
        mk(gA(quadA, 4), gA(quadA, 4), aA_s.at[2], aA_r.at[2], nz).start()
        mk(gB(pairA, 2), gB(pairA, 2), aB_s.at[3], aB_r.at[3], ny).start()
        mk(gB(pairA ^ 4, 2), gB(pairA ^ 4, 2),
           aB_s.at[4], aB_r.at[4], ny).start()
        mk(gC(ell), gC(ell), aC_s.at[3], aC_r.at[3], nx).start()
        mk(gC(ell ^ 2), gC(ell ^ 2), aC_s.at[4], aC_r.at[4], nx).start()
        mk(gC(ell ^ 4), gC(ell ^ 4), aC_s.at[5], aC_r.at[5], nx).start()
        mk(gC(ell ^ 6), gC(ell ^ 6), aC_s.at[6], aC_r.at[6], nx).start()
        out_ref[rows(pairA ^ 2, 2), A] = (
            gbuf[rows(pairA ^ 2, 2), A].astype(jnp.float32))
        out_ref[rows(ell ^ 1), B] = gbuf[rows(ell ^ 1), B].astype(jnp.float32)
        out_ref[rows(ell ^ 1 ^ 4), B] = (
            gbuf[rows(ell ^ 1 ^ 4), B].astype(jnp.float32))
        out_ref[rows(ell ^ 4), C] = gbuf[rows(ell ^ 4), C].astype(jnp.float32)
        out_ref[rows(ell ^ 6), C] = gbuf[rows(ell ^ 6), C].astype(jnp.float32)

        mk(gA(quadA ^ 4, 4), gA(quadA ^ 4, 4),
           aA_s.at[2], aA_r.at[2], nz).wait_recv()
        mk(gB(pairA ^ 2, 2), gB(pairA ^ 2, 2),
           aB_s.at[3], aB_r.at[3], ny).wait_recv()
        mk(gB(pairA ^ 2 ^ 4, 2), gB(pairA ^ 2 ^ 4, 2),
           aB_s.at[4], aB_r.at[4], ny).wait_recv()
        mk(gC(ell ^ 1), gC(ell ^ 1), aC_s.at[3], aC_r.at[3], nx).wait_recv()
        mk(gC(ell ^ 3), gC(ell ^ 3), aC_s.at[4], aC_r.at[4], nx).wait_recv()
        mk(gC(ell ^ 5), gC(ell ^ 5), aC_s.at[5], aC_r.at[5], nx).wait_recv()
        mk(gC(ell ^ 7), gC(ell ^ 7), aC_s.at[6], aC_r.at[6], nx).wait_recv()
        out_ref[rows(quadA ^ 4, 4), A] = (
            gbuf[rows(quadA ^ 4, 4), A].astype(jnp.float32))
        out_ref[rows(pairA ^ 2, 2), B] = (
            gbuf[rows(pairA ^ 2, 2), B].astype(jnp.float32))
        out_ref[rows(pairA ^ 2 ^ 4, 2), B] = (
            gbuf[rows(pairA ^ 2 ^ 4, 2), B].astype(jnp.float32))
        out_ref[rows(ell ^ 1), C] = gbuf[rows(ell ^ 1), C].astype(jnp.float32)
        out_ref[rows(ell ^ 3), C] = gbuf[rows(ell ^ 3), C].astype(jnp.float32)
        out_ref[rows(ell ^ 5), C] = gbuf[rows(ell ^ 5), C].astype(jnp.float32)
        out_ref[rows(ell ^ 7), C] = gbuf[rows(ell ^ 7), C].astype(jnp.float32)

        mk(rA0, rA0, A_s.at[0], A_r.at[0], nz).wait_send()
        mk(rA1, rA1, A_s.at[1], A_r.at[1], ny).wait_send()
        mk(rA2, rA2, A_s.at[2], A_r.at[2], nx).wait_send()
        mk(rB0.at[pl.ds(0, 2 * chunk)], rB0.at[pl.ds(0, 2 * chunk)],
           B_s.at[0], B_r.at[0], ny).wait_send()
        mk(rB0.at[pl.ds(0, 2 * chunk)], rB0.at[pl.ds(0, 2 * chunk)],
           B_s.at[1], B_r.at[1], ny).wait_send()
        mk(rB2, rB2, B_s.at[2], B_r.at[2], nx).wait_send()
        mk(rB2, rB2, B_s.at[3], B_r.at[3], nx).wait_send()
        mk(rB2, rB2, B_s.at[4], B_r.at[4], nz).wait_send()
        for i in range(7):
            mk(rC2, rC2, C_s.at[i], C_r.at[i], nx).wait_send()
        mk(gA(ell), gA(ell), aA_s.at[0], aA_r.at[0], nx).wait_send()
        mk(gA(pairA, 2), gA(pairA, 2), aA_s.at[1], aA_r.at[1], ny).wait_send()
        mk(gA(quadA, 4), gA(quadA, 4), aA_s.at[2], aA_r.at[2], nz).wait_send()
        mk(gB(ell), gB(ell), aB_s.at[0], aB_r.at[0], nz).wait_send()
        mk(gB(ell), gB(ell), aB_s.at[1], aB_r.at[1], nx).wait_send()
        mk(gB(ell), gB(ell), aB_s.at[2], aB_r.at[2], nx).wait_send()
        mk(gB(pairA, 2), gB(pairA, 2), aB_s.at[3], aB_r.at[3], ny).wait_send()
        mk(gB(pairA, 2), gB(pairA, 2), aB_s.at[4], aB_r.at[4], ny).wait_send()
        for i in range(7):
            mk(gC(ell), gC(ell), aC_s.at[i], aC_r.at[i], ny).wait_send()

    bf = jnp.bfloat16
    return pl.pallas_call(
        body,
        out_shape=jax.ShapeDtypeStruct((n_tok, h), jnp.float32),
        in_specs=[pl.BlockSpec(memory_space=pltpu.VMEM)] * 5,
        out_specs=pl.BlockSpec(memory_space=pltpu.VMEM),
        scratch_shapes=[
            pltpu.VMEM((n_tok, h), bf),
            pltpu.VMEM((n_tok, h), bf),
            pltpu.VMEM((n_tok, d), bf),
            pltpu.VMEM((4 * chunk, wA), bf),
            pltpu.VMEM((2 * chunk, wA), bf),
            pltpu.VMEM((chunk, wA), bf),
            pltpu.VMEM((4 * chunk, wB), bf),
            pltpu.VMEM((2 * chunk, wB), bf),
            pltpu.VMEM((chunk, wB), bf),
            pltpu.VMEM((4 * chunk, wC), bf),
            pltpu.VMEM((2 * chunk, wC), bf),
            pltpu.VMEM((chunk, wC), bf),
            pltpu.VMEM((n_tok, 1), jnp.float32),
            pltpu.SemaphoreType.DMA((3,)),
            pltpu.SemaphoreType.DMA((3,)),
            pltpu.SemaphoreType.DMA((5,)),
            pltpu.SemaphoreType.DMA((5,)),
            pltpu.SemaphoreType.DMA((7,)),
            pltpu.SemaphoreType.DMA((7,)),
            pltpu.SemaphoreType.DMA((3,)),
            pltpu.SemaphoreType.DMA((3,)),
            pltpu.SemaphoreType.DMA((5,)),
            pltpu.SemaphoreType.DMA((5,)),
            pltpu.SemaphoreType.DMA((7,)),
            pltpu.SemaphoreType.DMA((7,)),
        ],
        compiler_params=pltpu.CompilerParams(collective_id=0),
    )(x, router_W, route_idx, expert_W, shared_W)


# device time: 43289 ns/iter; 1.5601x vs baseline; 1.0186x over previous
import jax
import jax.numpy as jnp
from jax import lax
from jax.experimental import pallas as pl
from jax.experimental.pallas import tpu as pltpu

N_DEV = 8


def kernel(x, router_W, route_idx, expert_W, shared_W):
    n_tok, d = x.shape
    n_exp = router_W.shape[1]
    e_loc, _, h = expert_W.shape
    chunk = n_tok // N_DEV
    wA = 384
    wB = 384
    wC = h - wA - wB

    def body(x_ref, rw_ref, idx_ref, ew_ref, sw_ref, out_ref,
             partial_ref, gbuf, x16, rA0, rA1, rA2, rB0, rB1, rB2,
             rC0, rC1, rC2, psel_ref,
             A_s, A_r, B_s, B_r, C_s, C_r, aA_s, aA_r, aB_s, aB_r,
             aC_s, aC_r):
        my = lax.axis_index("i")
        ell = my ^ ((my >> 1) & 1)
        b1 = ell & 1
        b2 = (ell >> 1) & 1
        b4 = (ell >> 2) & 1

        def logi(l):
            return l ^ ((l >> 1) & 1)

        nx = logi(ell ^ 1)
        ny = logi(ell ^ 2)
        nz = logi(ell ^ 4)

        def rows(c, n=1):
            return pl.ds(c * chunk, n * chunk)

        A = pl.ds(0, wA)
        B = pl.ds(wA, wB)
        C = pl.ds(wA + wB, wC)

        def mk(src, dst, ssem, rsem, dev):
            return pltpu.make_async_remote_copy(
                src_ref=src, dst_ref=dst, send_sem=ssem, recv_sem=rsem,
                device_id=(dev,), device_id_type=pl.DeviceIdType.MESH)

        xv = x_ref[:, :]
        x16[:, :] = xv.astype(jnp.bfloat16)

        scores = jnp.dot(xv, rw_ref[:, :], preferred_element_type=jnp.float32)
        m = jnp.max(scores, axis=-1, keepdims=True)
        p = jnp.exp(scores - m)
        probs = p / jnp.sum(p, axis=-1, keepdims=True)
        ridx = idx_ref[:, 0:1]
        e_ids = lax.broadcasted_iota(jnp.int32, (n_tok, n_exp), 1)
        p_sel = jnp.sum(jnp.where(e_ids == ridx, probs, 0.0),
                        axis=1, keepdims=True)
        psel_ref[:, :] = p_sel

        def comp(rs_chunks, n_chunks, col0, w):
            rws = pl.ds(rs_chunks * chunk, n_chunks * chunk)
            xb = x16[rws, :]
            rb = idx_ref[rws, 0:1]
            pb = psel_ref[rws, :]
            accu = jnp.zeros((n_chunks * chunk, w), jnp.float32)
            for k in range(e_loc):
                wk = ew_ref[k, :, col0:col0 + w].astype(jnp.bfloat16)
                ck = jnp.where(rb == my * e_loc + k, pb, 0.0)
                accu = accu + ck * jnp.dot(
                    xb, wk, preferred_element_type=jnp.float32)
            partial_ref[rws, col0:col0 + w] = accu.astype(jnp.bfloat16)

        kA0 = b4 * 4
        sA0 = (1 - b4) * 4
        kA1 = kA0 + b2 * 2
        sA1 = kA0 + (1 - b2) * 2
        sA2 = kA1 + (1 - b1)
        kB0a = b2 * 2
        kB0b = 4 + b2 * 2
        sB0a = (1 - b2) * 2
        sB0b = 4 + (1 - b2) * 2
        kB1a = kB0a + b1
        kB1b = kB0b + b1
        sB1a = kB0a + (1 - b1)
        sB1b = kB0b + (1 - b1)
        sB2 = (1 - b4) * 4 + b2 * 2 + b1
        sC2 = b1 + 4 * b4 + 2 * (1 - b2)

        barrier_sem = pltpu.get_barrier_semaphore()
        for nbr in (nx, ny, nz):
            pl.semaphore_signal(barrier_sem, inc=1, device_id=(nbr,),
                                device_id_type=pl.DeviceIdType.MESH)
        pl.semaphore_wait(barrier_sem, 3)

        comp(sA0, 2, 0, wA)
        mk(partial_ref.at[rows(sA0, 2), A], rA0.at[pl.ds(0, 2 * chunk)],
           A_s.at[0], A_r.at[0], nz).start()
        comp(sB0a, 2, wA, wB)
        mk(partial_ref.at[rows(sB0a, 2), B], rB0.at[pl.ds(0, 2 * chunk)],
           B_s.at[0], B_r.at[0], ny).start()
        comp(sA0 + 2, 2, 0, wA)
        mk(partial_ref.at[rows(sA0 + 2, 2), A],
           rA0.at[pl.ds(2 * chunk, 2 * chunk)],
           A_s.at[1], A_r.at[1], nz).start()
        comp(sB0b, 2, wA, wB)
        mk(partial_ref.at[rows(sB0b, 2), B], rB0.at[pl.ds(2 * chunk, 2 * chunk)],
           B_s.at[1], B_r.at[1], ny).start()
        for i in range(4):
            comp((1 - b1) + 2 * i, 1, wA + wB, wC)
            mk(partial_ref.at[rows((1 - b1) + 2 * i), C],
               rC0.at[pl.ds(i * chunk, chunk)],
               C_s.at[i], C_r.at[i], nx).start()

        comp(kA0, 4, 0, wA)
        comp(kB0a, 2, wA, wB)
        comp(kB0b, 2, wA, wB)
        for i in range(4):
            comp(b1 + 2 * i, 1, wA + wB, wC)

        mk(rA0.at[pl.ds(0, 2 * chunk)], rA0.at[pl.ds(0, 2 * chunk)],
           A_s.at[0], A_r.at[0], nz).wait_recv()
        mk(rA0.at[pl.ds(2 * chunk, 2 * chunk)],
           rA0.at[pl.ds(2 * chunk, 2 * chunk)],
           A_s.at[1], A_r.at[1], nz).wait_recv()
        partial_ref[rows(sA1, 2), A] = (
            partial_ref[rows(sA1, 2), A]
            + rA0[pl.ds((1 - b2) * 2 * chunk, 2 * chunk), :])
        mk(partial_ref.at[rows(sA1, 2), A], rA1,
           A_s.at[2], A_r.at[2], ny).start()
        partial_ref[rows(kA1, 2), A] = (
            partial_ref[rows(kA1, 2), A]
            + rA0[pl.ds(b2 * 2 * chunk, 2 * chunk), :])

        mk(rB0.at[pl.ds(0, 2 * chunk)], rB0.at[pl.ds(0, 2 * chunk)],
           B_s.at[0], B_r.at[0], ny).wait_recv()
        mk(rB0.at[pl.ds(2 * chunk, 2 * chunk)], rB0.at[pl.ds(2 * chunk, 2 * chunk)],
           B_s.at[1], B_r.at[1], ny).wait_recv()
        partial_ref[rows(sB1a), B] = (
            partial_ref[rows(sB1a), B]
            + rB0[pl.ds((1 - b1) * chunk, chunk), :])
        partial_ref[rows(sB1b), B] = (
            partial_ref[rows(sB1b), B]
            + rB0[pl.ds(2 * chunk + (1 - b1) * chunk, chunk), :])
        mk(partial_ref.at[rows(sB1a), B], rB1.at[pl.ds(0, chunk)],
           B_s.at[2], B_r.at[2], nx).start()
        mk(partial_ref.at[rows(sB1b), B], rB1.at[pl.ds(chunk, chunk)],
           B_s.at[3], B_r.at[3], nx).start()
        partial_ref[rows(kB1a), B] = (
            partial_ref[rows(kB1a), B] + rB0[pl.ds(b1 * chunk, chunk), :])
        partial_ref[rows(kB1b), B] = (
            partial_ref[rows(kB1b), B]
            + rB0[pl.ds(2 * chunk + b1 * chunk, chunk), :])

        for i in range(4):
            mk(rC0.at[pl.ds(i * chunk, chunk)], rC0.at[pl.ds(i * chunk, chunk)],
               C_s.at[i], C_r.at[i], nx).wait_recv()
        for j in range(2):
            i = (1 - b4) * 2 + j
            cc = b1 + 2 * i
            partial_ref[rows(cc), C] = (
                partial_ref[rows(cc), C] + rC0[pl.ds(i * chunk, chunk), :])
            mk(partial_ref.at[rows(cc), C], rC1.at[pl.ds(j * chunk, chunk)],
               C_s.at[4 + j], C_r.at[4 + j], nz).start()
        for j in range(2):
            i = b4 * 2 + j
            cc = b1 + 2 * i
            partial_ref[rows(cc), C] = (
                partial_ref[rows(cc), C] + rC0[pl.ds(i * chunk, chunk), :])

        sh = jnp.dot(x_ref[rows(ell), :], sw_ref[:, :],
                     preferred_element_type=jnp.float32)

        mk(rA1, rA1, A_s.at[2], A_r.at[2], ny).wait_recv()
        partial_ref[rows(sA2), A] = (
            partial_ref[rows(sA2), A]
            + rA1[pl.ds((1 - b1) * chunk, chunk), :])
        mk(partial_ref.at[rows(sA2), A], rA2,
           A_s.at[3], A_r.at[3], nx).start()
        partial_ref[rows(ell), A] = (
            partial_ref[rows(ell), A] + rA1[pl.ds(b1 * chunk, chunk), :])

        mk(rB1.at[pl.ds(0, chunk)], rB1.at[pl.ds(0, chunk)],
           B_s.at[2], B_r.at[2], nx).wait_recv()
        mk(rB1.at[pl.ds(chunk, chunk)], rB1.at[pl.ds(chunk, chunk)],
           B_s.at[3], B_r.at[3], nx).wait_recv()
        partial_ref[rows(sB2), B] = (
            partial_ref[rows(sB2), B]
            + rB1[pl.ds((1 - b4) * chunk, chunk), :])
        mk(partial_ref.at[rows(sB2), B], rB2,
           B_s.at[4], B_r.at[4], nz).start()
        partial_ref[rows(ell), B] = (
            partial_ref[rows(ell), B] + rB1[pl.ds(b4 * chunk, chunk), :])

        mk(rC1.at[pl.ds(0, chunk)], rC1.at[pl.ds(0, chunk)],
           C_s.at[4], C_r.at[4], nz).wait_recv()
        mk(rC1.at[pl.ds(chunk, chunk)], rC1.at[pl.ds(chunk, chunk)],
           C_s.at[5], C_r.at[5], nz).wait_recv()
        partial_ref[rows(sC2), C] = (
            partial_ref[rows(sC2), C]
            + rC1[pl.ds((1 - b2) * chunk, chunk), :])
        mk(partial_ref.at[rows(sC2), C], rC2,
           C_s.at[6], C_r.at[6], ny).start()
        partial_ref[rows(ell), C] = (
            partial_ref[rows(ell), C] + rC1[pl.ds(b2 * chunk, chunk), :])

        mk(rA2, rA2, A_s.at[3], A_r.at[3], nx).wait_recv()
        mk(rB2, rB2, B_s.at[4], B_r.at[4], nz).wait_recv()
        mk(rC2, rC2, C_s.at[6], C_r.at[6], ny).wait_recv()
        vA = (partial_ref[rows(ell), A] + rA2[:, :]).astype(jnp.float32) \
            + sh[:, 0:wA]
        vB = (partial_ref[rows(ell), B] + rB2[:, :]).astype(jnp.float32) \
            + sh[:, wA:wA + wB]
        vC = (partial_ref[rows(ell), C] + rC2[:, :]).astype(jnp.float32) \
            + sh[:, wA + wB:h]
        gbuf[rows(ell), A] = vA.astype(jnp.bfloat16)
        gbuf[rows(ell), B] = vB.astype(jnp.bfloat16)
        gbuf[rows(ell), C] = vC.astype(jnp.bfloat16)

        pairA = ell & ~1
        quadA = ell & ~3

        def gA(c, n=1):
            return gbuf.at[rows(c, n), A]

        def gB(c, n=1):
            return gbuf.at[rows(c, n), B]

        def gC(c, n=1):
            return gbuf.at[rows(c, n), C]

        mk(gA(ell), gA(ell), aA_s.at[0], aA_r.at[0], nx).start()
        mk(gB(ell), gB(ell), aB_s.at[0], aB_r.at[0], nz).start()
        mk(gC(ell), gC(ell), aC_s.at[0], aC_r.at[0], ny).start()
        out_ref[rows(ell), A] = vA
        out_ref[rows(ell), B] = vB
        out_ref[rows(ell), C] = vC

        mk(gA(ell ^ 1), gA(ell ^ 1), aA_s.at[0], aA_r.at[0], nx).wait_recv()
        mk(gB(ell ^ 4), gB(ell ^ 4), aB_s.at[0], aB_r.at[0], nz).wait_recv()
        mk(gC(ell ^ 2), gC(ell ^ 2), aC_s.at[0], aC_r.at[0], ny).wait_recv()
        mk(gA(pairA, 2), gA(pairA, 2), aA_s.at[1], aA_r.at[1], ny).start()
        mk(gB(ell), gB(ell), aB_s.at[1], aB_r.at[1], nx).start()
        mk(gB(ell ^ 4), gB(ell ^ 4), aB_s.at[2], aB_r.at[2], nx).start()
        mk(gC(ell), gC(ell), aC_s.at[1], aC_r.at[1], nz).start()
        mk(gC(ell ^ 2), gC(ell ^ 2), aC_s.at[2], aC_r.at[2], nz).start()
        out_ref[rows(ell ^ 1), A] = gbuf[rows(ell ^ 1), A].astype(jnp.float32)
        out_ref[rows(ell ^ 4), B] = gbuf[rows(ell ^ 4), B].astype(jnp.float32)
        out_ref[rows(ell ^ 2), C] = gbuf[rows(ell ^ 2), C].astype(jnp.float32)

        mk(gA(pairA ^ 2, 2), gA(pairA ^ 2, 2),
           aA_s.at[1], aA_r.at[1], ny).wait_recv()
        mk(gB(ell ^ 1), gB(ell ^ 1), aB_s.at[1], aB_r.at[1], nx).wait_recv()
        mk(gB(ell ^ 1 ^ 4), gB(ell ^ 1 ^ 4),
           aB_s.at[2], aB_r.at[2], nx).wait_recv()
        mk(gC(ell ^ 4), gC(ell ^ 4), aC_s.at[1], aC_r.at[1], nz).wait_recv()
        mk(gC(ell ^ 4 ^ 2), gC(ell ^ 4 ^ 2),
           aC_s.at[2], aC_r.at[2], nz).wait_recv()
        mk(gA(quadA, 4), gA(quadA, 4), aA_s.at[2], aA_r.at[2], nz).start()
        mk(gB(pairA, 2), gB(pairA, 2), aB_s.at[3], aB_r.at[3], ny).start()
        mk(gB(pairA ^ 4, 2), gB(pairA ^ 4, 2),
           aB_s.at[4], aB_r.at[4], ny).start()
        mk(gC(ell), gC(ell), aC_s.at[3], aC_r.at[3], nx).start()
        mk(gC(ell ^ 2), gC(ell ^ 2), aC_s.at[4], aC_r.at[4], nx).start()
        mk(gC(ell ^ 4), gC(ell ^ 4), aC_s.at[5], aC_r.at[5], nx).start()
        mk(gC(ell ^ 6), gC(ell ^ 6), aC_s.at[6], aC_r.at[6], nx).start()
        out_ref[rows(pairA ^ 2, 2), A] = (
            gbuf[rows(pairA ^ 2, 2), A].astype(jnp.float32))
        out_ref[rows(ell ^ 1), B] = gbuf[rows(ell ^ 1), B].astype(jnp.float32)
        out_ref[rows(ell ^ 1 ^ 4), B] = (
            gbuf[rows(ell ^ 1 ^ 4), B].astype(jnp.float32))
        out_ref[rows(ell ^ 4), C] = gbuf[rows(ell ^ 4), C].astype(jnp.float32)
        out_ref[rows(ell ^ 6), C] = gbuf[rows(ell ^ 6), C].astype(jnp.float32)

        mk(gC(ell ^ 1), gC(ell ^ 1), aC_s.at[3], aC_r.at[3], nx).wait_recv()
        out_ref[rows(ell ^ 1), C] = gbuf[rows(ell ^ 1), C].astype(jnp.float32)
        mk(gC(ell ^ 3), gC(ell ^ 3), aC_s.at[4], aC_r.at[4], nx).wait_recv()
        out_ref[rows(ell ^ 3), C] = gbuf[rows(ell ^ 3), C].astype(jnp.float32)
        mk(gB(pairA ^ 2, 2), gB(pairA ^ 2, 2),
           aB_s.at[3], aB_r.at[3], ny).wait_recv()
        out_ref[rows(pairA ^ 2, 2), B] = (
            gbuf[rows(pairA ^ 2, 2), B].astype(jnp.float32))
        mk(gC(ell ^ 5), gC(ell ^ 5), aC_s.at[5], aC_r.at[5], nx).wait_recv()
        out_ref[rows(ell ^ 5), C] = gbuf[rows(ell ^ 5), C].astype(jnp.float32)
        mk(gB(pairA ^ 2 ^ 4, 2), gB(pairA ^ 2 ^ 4, 2),
           aB_s.at[4], aB_r.at[4], ny).wait_recv()
        out_ref[rows(pairA ^ 2 ^ 4, 2), B] = (
            gbuf[rows(pairA ^ 2 ^ 4, 2), B].astype(jnp.float32))
        mk(gC(ell ^ 7), gC(ell ^ 7), aC_s.at[6], aC_r.at[6], nx).wait_recv()
        out_ref[rows(ell ^ 7), C] = gbuf[rows(ell ^ 7), C].astype(jnp.float32)
        mk(gA(quadA ^ 4, 4), gA(quadA ^ 4, 4),
           aA_s.at[2], aA_r.at[2], nz).wait_recv()
        out_ref[rows(quadA ^ 4, 4), A] = (
            gbuf[rows(quadA ^ 4, 4), A].astype(jnp.float32))

        mk(rA1, rA1, A_s.at[0], A_r.at[0], nz).wait_send()
        mk(rA1, rA1, A_s.at[1], A_r.at[1], nz).wait_send()
        mk(rA1, rA1, A_s.at[2], A_r.at[2], ny).wait_send()
        mk(rA2, rA2, A_s.at[3], A_r.at[3], nx).wait_send()
        mk(rB0.at[pl.ds(0, 2 * chunk)], rB0.at[pl.ds(0, 2 * chunk)],
           B_s.at[0], B_r.at[0], ny).wait_send()
        mk(rB0.at[pl.ds(0, 2 * chunk)], rB0.at[pl.ds(0, 2 * chunk)],
           B_s.at[1], B_r.at[1], ny).wait_send()
        mk(rB2, rB2, B_s.at[2], B_r.at[2], nx).wait_send()
        mk(rB2, rB2, B_s.at[3], B_r.at[3], nx).wait_send()
        mk(rB2, rB2, B_s.at[4], B_r.at[4], nz).wait_send()
        for i in range(7):
            mk(rC2, rC2, C_s.at[i], C_r.at[i], nx).wait_send()
        mk(gA(ell), gA(ell), aA_s.at[0], aA_r.at[0], nx).wait_send()
        mk(gA(pairA, 2), gA(pairA, 2), aA_s.at[1], aA_r.at[1], ny).wait_send()
        mk(gA(quadA, 4), gA(quadA, 4), aA_s.at[2], aA_r.at[2], nz).wait_send()
        mk(gB(ell), gB(ell), aB_s.at[0], aB_r.at[0], nz).wait_send()
        mk(gB(ell), gB(ell), aB_s.at[1], aB_r.at[1], nx).wait_send()
        mk(gB(ell), gB(ell), aB_s.at[2], aB_r.at[2], nx).wait_send()
        mk(gB(pairA, 2), gB(pairA, 2), aB_s.at[3], aB_r.at[3], ny).wait_send()
        mk(gB(pairA, 2), gB(pairA, 2), aB_s.at[4], aB_r.at[4], ny).wait_send()
        for i in range(7):
            mk(gC(ell), gC(ell), aC_s.at[i], aC_r.at[i], ny).wait_send()

    bf = jnp.bfloat16
    return pl.pallas_call(
        body,
        out_shape=jax.ShapeDtypeStruct((n_tok, h), jnp.float32),
        in_specs=[pl.BlockSpec(memory_space=pltpu.VMEM)] * 5,
        out_specs=pl.BlockSpec(memory_space=pltpu.VMEM),
        scratch_shapes=[
            pltpu.VMEM((n_tok, h), bf),
            pltpu.VMEM((n_tok, h), bf),
            pltpu.VMEM((n_tok, d), bf),
            pltpu.VMEM((4 * chunk, wA), bf),
            pltpu.VMEM((2 * chunk, wA), bf),
            pltpu.VMEM((chunk, wA), bf),
            pltpu.VMEM((4 * chunk, wB), bf),
            pltpu.VMEM((2 * chunk, wB), bf),
            pltpu.VMEM((chunk, wB), bf),
            pltpu.VMEM((4 * chunk, wC), bf),
            pltpu.VMEM((2 * chunk, wC), bf),
            pltpu.VMEM((chunk, wC), bf),
            pltpu.VMEM((n_tok, 1), jnp.float32),
            pltpu.SemaphoreType.DMA((4,)),
            pltpu.SemaphoreType.DMA((4,)),
            pltpu.SemaphoreType.DMA((5,)),
            pltpu.SemaphoreType.DMA((5,)),
            pltpu.SemaphoreType.DMA((7,)),
            pltpu.SemaphoreType.DMA((7,)),
            pltpu.SemaphoreType.DMA((3,)),
            pltpu.SemaphoreType.DMA((3,)),
            pltpu.SemaphoreType.DMA((5,)),
            pltpu.SemaphoreType.DMA((5,)),
            pltpu.SemaphoreType.DMA((7,)),
            pltpu.SemaphoreType.DMA((7,)),
        ],
        compiler_params=pltpu.CompilerParams(collective_id=0),
    )(x, router_W, route_idx, expert_W, shared_W)
